# baseline (device time: 31133 ns/iter reference)
import jax
import jax.numpy as jnp
from jax import lax
from jax.experimental import pallas as pl
from jax.experimental.pallas import tpu as pltpu

N_DEV = 4


def kernel(Q, K, V):
    b, s, h, d = Q.shape
    bh = b * h
    scale = d ** -0.5

    q3 = (
        (jnp.transpose(Q, (0, 2, 1, 3)) * scale)
        .reshape(bh, s, d)
        .astype(jnp.bfloat16)
    )

    kt = jnp.transpose(K, (0, 2, 3, 1)).reshape(bh, d, s)
    vn = jnp.transpose(V, (0, 2, 1, 3)).reshape(bh, s, d)

    def enc_scale(sc):
        q14 = jnp.round(sc * 65536.0).astype(jnp.int32)
        return (q14 // 127).astype(jnp.int8), (q14 % 127).astype(jnp.int8)

    ks = jnp.max(jnp.abs(kt), axis=1, keepdims=True) / 127.0
    k8 = jnp.clip(jnp.round(kt / ks), -127, 127).astype(jnp.int8)
    ka, kb2 = enc_scale(ks)

    vs = jnp.max(jnp.abs(vn), axis=2, keepdims=True) / 127.0
    v8 = jnp.clip(jnp.round(vn / vs), -127, 127).astype(jnp.int8)
    va, vb2 = enc_scale(jnp.transpose(vs, (0, 2, 1)))

    k3 = jnp.concatenate([k8, ka, kb2, va, vb2], axis=1)
    dk = d + 4

    def body(q_ref, k_ref, v_ref, out_ref, kbuf, vbuf, ss, rs):
        my = lax.axis_index("i")
        left = (my - 1) % N_DEV
        right = (my + 1) % N_DEV

        barrier = pltpu.get_barrier_semaphore()
        for nbr in (left, right):
            pl.semaphore_signal(
                barrier, inc=1, device_id=(nbr,),
                device_id_type=pl.DeviceIdType.MESH,
            )
        pl.semaphore_wait(barrier, 2)

        kbuf[0] = k_ref[...]
        vbuf[0] = v_ref[...]

        def rdma(src, dst, i, dev):
            return pltpu.make_async_remote_copy(
                src_ref=src, dst_ref=dst,
                send_sem=ss.at[i], recv_sem=rs.at[i],
                device_id=(dev,), device_id_type=pl.DeviceIdType.MESH,
            )

        k_r = rdma(kbuf.at[0], kbuf.at[1], 0, right)
        v_r = rdma(vbuf.at[0], vbuf.at[1], 1, right)
        k_l = rdma(kbuf.at[0], kbuf.at[2], 2, left)
        v_l = rdma(vbuf.at[0], vbuf.at[2], 3, left)
        for r in (k_r, v_r, k_l, v_l):
            r.start()

        def sphase(slot, ls, c0, c1):
            ps, new_ls = [], []
            for i in range(bh):
                ka_ = kbuf[slot, i, d:d + 1, c0:c1].astype(jnp.float32)
                kb_ = kbuf[slot, i, d + 1:d + 2, c0:c1].astype(jnp.float32)
                va_ = kbuf[slot, i, d + 2:d + 3, c0:c1].astype(jnp.float32)
                vb_ = kbuf[slot, i, d + 3:d + 4, c0:c1].astype(jnp.float32)
                ksr = (ka_ * 127.0 + kb_) * (1.0 / 65536.0)
                vsr = (va_ * 127.0 + vb_) * (1.0 / 65536.0)
                s_int = lax.dot_general(
                    q_ref[i], kbuf[slot, i, 0:d, c0:c1].astype(jnp.bfloat16),
                    (((1,), (0,)), ((), ())),
                    preferred_element_type=jnp.float32,
                )
                p = jnp.exp(s_int * ksr)
                lsum = jnp.sum(p, axis=1, keepdims=True)
                ps.append((p * vsr).astype(jnp.bfloat16))
                new_ls.append(lsum if ls is None else ls[i] + lsum)
            return ps, new_ls

        def pvphase(slot, ps, accs, c0, c1):
            new_accs = []
            for i in range(bh):
                pv = lax.dot_general(
                    ps[i], vbuf[slot, i, c0:c1, :].astype(jnp.bfloat16),
                    (((1,), (0,)), ((), ())),
                    preferred_element_type=jnp.float32,
                )
                new_accs.append(pv if accs is None else accs[i] + pv)
            return new_accs

        sh = s // 2

        p0, ls = sphase(0, None, 0, s)
        accs = pvphase(0, p0, None, 0, s)

        k_r.wait_recv()
        k_fa = rdma(kbuf.at[1, :, :, 0:sh], kbuf.at[3, :, :, 0:sh], 4, right)
        k_fb = rdma(kbuf.at[1, :, :, sh:s], kbuf.at[3, :, :, sh:s], 5, right)
        k_fa.start()
        k_fb.start()

        p1, ls = sphase(1, ls, 0, s)
        k_l.wait_recv()
        p2, ls = sphase(2, ls, 0, s)

        v_r.wait_recv()
        accs = pvphase(1, p1, accs, 0, s)
        v_l.wait_recv()
        v_fa = rdma(vbuf.at[2, :, 0:sh, :], vbuf.at[3, :, 0:sh, :], 6, left)
        v_fb = rdma(vbuf.at[2, :, sh:s, :], vbuf.at[3, :, sh:s, :], 7, left)
        v_fa.start()
        v_fb.start()
        accs = pvphase(2, p2, accs, 0, s)

        k_fa.wait_recv()
        v_fa.wait_recv()
        p3a, ls = sphase(3, ls, 0, sh)
        accs = pvphase(3, p3a, accs, 0, sh)
        k_fb.wait_recv()
        v_fb.wait_recv()
        p3b, ls = sphase(3, ls, sh, s)
        accs = pvphase(3, p3b, accs, sh, s)

        for i in range(bh):
            out_ref[i] = accs[i] / ls[i]

        for r in (k_r, v_r, k_l, v_l, k_fa, k_fb, v_fa, v_fb):
            r.wait_send()

    params_cls = getattr(pltpu, "CompilerParams", None) or pltpu.TPUCompilerParams
    out = pl.pallas_call(
        body,
        out_shape=jax.ShapeDtypeStruct((bh, s, d), jnp.float32),
        in_specs=[pl.BlockSpec(memory_space=pltpu.VMEM)] * 3,
        out_specs=pl.BlockSpec(memory_space=pltpu.VMEM),
        scratch_shapes=[
            pltpu.VMEM((N_DEV, bh, dk, s), jnp.int8),
            pltpu.VMEM((N_DEV, bh, s, d), jnp.int8),
            pltpu.SemaphoreType.DMA((8,)),
            pltpu.SemaphoreType.DMA((8,)),
        ],
        compiler_params=params_cls(collective_id=0),
    )(q3, k3, v8)

    return out.reshape(b, h, s, d).transpose(0, 2, 1, 3)


# device time: 30034 ns/iter; 1.0366x vs baseline; 1.0366x over previous
import jax
import jax.numpy as jnp
from jax import lax
from jax.experimental import pallas as pl
from jax.experimental.pallas import tpu as pltpu

N_DEV = 4


def kernel(Q, K, V):
    b, s, h, d = Q.shape
    bh = b * h
    scale = d ** -0.5

    q3 = (
        (jnp.transpose(Q, (0, 2, 1, 3)) * scale)
        .reshape(bh, s, d)
        .astype(jnp.bfloat16)
    )

    kt = jnp.transpose(K, (0, 2, 3, 1)).reshape(bh, d, s)
    vn = jnp.transpose(V, (0, 2, 1, 3)).reshape(bh, s, d)

    def enc_scale(sc):
        q14 = jnp.round(sc * 65536.0).astype(jnp.int32)
        return (q14 // 127).astype(jnp.int8), (q14 % 127).astype(jnp.int8)

    ks = jnp.max(jnp.abs(kt), axis=1, keepdims=True) / 127.0
    k8 = jnp.clip(jnp.round(kt / ks), -127, 127).astype(jnp.int8)
    ka, kb2 = enc_scale(ks)

    vs = jnp.max(jnp.abs(vn), axis=2, keepdims=True) / 127.0
    v8 = jnp.clip(jnp.round(vn / vs), -127, 127).astype(jnp.int8)
    va, vb2 = enc_scale(jnp.transpose(vs, (0, 2, 1)))

    k3 = jnp.concatenate([k8, ka, kb2, va, vb2], axis=1)
    dk = d + 4

    def body(q_ref, k_ref, v_ref, out_ref, kbuf, vbuf, ss, rs):
        my = lax.axis_index("i")
        left = (my - 1) % N_DEV
        right = (my + 1) % N_DEV

        barrier = pltpu.get_barrier_semaphore()
        for nbr in (left, right):
            pl.semaphore_signal(
                barrier, inc=1, device_id=(nbr,),
                device_id_type=pl.DeviceIdType.MESH,
            )
        pl.semaphore_wait(barrier, 2)

        kbuf[0] = k_ref[...]
        vbuf[0] = v_ref[...]

        def rdma(src, dst, i, dev):
            return pltpu.make_async_remote_copy(
                src_ref=src, dst_ref=dst,
                send_sem=ss.at[i], recv_sem=rs.at[i],
                device_id=(dev,), device_id_type=pl.DeviceIdType.MESH,
            )

        k_r = rdma(kbuf.at[0], kbuf.at[1], 0, right)
        v_r = rdma(vbuf.at[0], vbuf.at[1], 1, right)
        k_l = rdma(kbuf.at[0], kbuf.at[2], 2, left)
        v_l = rdma(vbuf.at[0], vbuf.at[2], 3, left)
        for r in (k_r, v_r, k_l, v_l):
            r.start()

        def update(slot, accs, ls):
            new_accs, new_ls = [], []
            for i in range(bh):
                ka_ = kbuf[slot, i, d:d + 1, :].astype(jnp.float32)
                kb_ = kbuf[slot, i, d + 1:d + 2, :].astype(jnp.float32)
                va_ = kbuf[slot, i, d + 2:d + 3, :].astype(jnp.float32)
                vb_ = kbuf[slot, i, d + 3:d + 4, :].astype(jnp.float32)
                ksr = (ka_ * 127.0 + kb_) * (1.0 / 65536.0)
                vsr = (va_ * 127.0 + vb_) * (1.0 / 65536.0)
                s_int = lax.dot_general(
                    q_ref[i], kbuf[slot, i, 0:d, :].astype(jnp.bfloat16),
                    (((1,), (0,)), ((), ())),
                    preferred_element_type=jnp.float32,
                )
                p = jnp.exp(s_int * ksr)
                lsum = jnp.sum(p, axis=1, keepdims=True)
                pv = lax.dot_general(
                    (p * vsr).astype(jnp.bfloat16),
                    vbuf[slot, i].astype(jnp.bfloat16),
                    (((1,), (0,)), ((), ())),
                    preferred_element_type=jnp.float32,
                )
                if accs is None:
                    new_accs.append(pv)
                    new_ls.append(lsum)
                else:
                    new_accs.append(accs[i] + pv)
                    new_ls.append(ls[i] + lsum)
            return new_accs, new_ls

        accs, ls = update(0, None, None)

        k_r.wait_recv()
        k_f = rdma(kbuf.at[1], kbuf.at[3], 4, right)
        k_f.start()
        v_l.wait_recv()
        v_f = rdma(vbuf.at[2], vbuf.at[3], 5, left)
        v_f.start()

        v_r.wait_recv()
        accs, ls = update(1, accs, ls)
        k_l.wait_recv()
        accs, ls = update(2, accs, ls)

        k_f.wait_recv()
        v_f.wait_recv()
        accs, ls = update(3, accs, ls)

        for i in range(bh):
            out_ref[i] = accs[i] / ls[i]

        for r in (k_r, v_r, k_l, v_l, k_f, v_f):
            r.wait_send()

    params_cls = getattr(pltpu, "CompilerParams", None) or pltpu.TPUCompilerParams
    out = pl.pallas_call(
        body,
        out_shape=jax.ShapeDtypeStruct((bh, s, d), jnp.float32),
        in_specs=[pl.BlockSpec(memory_space=pltpu.VMEM)] * 3,
        out_specs=pl.BlockSpec(memory_space=pltpu.VMEM),
        scratch_shapes=[
            pltpu.VMEM((N_DEV, bh, dk, s), jnp.int8),
            pltpu.VMEM((N_DEV, bh, s, d), jnp.int8),
            pltpu.SemaphoreType.DMA((6,)),
            pltpu.SemaphoreType.DMA((6,)),
        ],
        compiler_params=params_cls(collective_id=0),
    )(q3, k3, v8)

    return out.reshape(b, h, s, d).transpose(0, 2, 1, 3)
